# Initial kernel scaffold; baseline (speedup 1.0000x reference)
import jax
import jax.numpy as jnp
from jax.experimental import pallas as pl


def kernel(embeddings_int, embeddings_pop, user, item_p, item_n, mask, graph):
    raise NotImplementedError("write your pallas kernel here")


# speedup vs baseline: 3.0401x; 3.0401x over previous
import functools

import jax
import jax.numpy as jnp
from jax import lax
from jax.experimental import pallas as pl
from jax.experimental.pallas import tpu as pltpu
from jax.experimental.pallas import tpu_sc as plsc

_NU = 25000
_N = 50000
_E = 800000
_B = 4096
_BN = 1000
_NB = _N // _BN
_NPT = _N // 16
_ZC = 3200
_NP = 16 * _ZC
_WC = 3128
_WL = _N - 15 * _WC

@functools.cache
def _sc_mesh():
    return plsc.VectorSubcoreMesh(core_axis_name="c", subcore_axis_name="s")


def _zero_1d(acc, zbuf, s):
    pltpu.sync_copy(zbuf, acc.at[pl.ds(s * _ZC, _ZC)])


def _wb_1d(acc, out, c, s, wbuf):
    off = s * _ZC
    pltpu.sync_copy(acc.at[pl.ds(off, _ZC)], wbuf)
    pltpu.sync_copy(wbuf, out.at[pl.ds(c * _NP + off, _ZC)])


def _sc_stats(dst, item_all, user_idx, zeros1, ones128):
    @functools.partial(
        pl.kernel,
        out_type=[jax.ShapeDtypeStruct((2 * _NP,), jnp.float32)] * 3,
        mesh=_sc_mesh(),
        compiler_params=pltpu.CompilerParams(use_tc_tiling_on_sc=False),
        scratch_types=[
            pltpu.VMEM_SHARED((_NP,), jnp.float32),
            pltpu.VMEM_SHARED((_NP,), jnp.float32),
            pltpu.VMEM_SHARED((_NP,), jnp.float32),
            pltpu.VMEM((1, 128), jnp.int32),
            pltpu.VMEM((128,), jnp.float32),
            pltpu.VMEM((_ZC,), jnp.float32),
        ],
    )
    def k(dst_h, item_h, user_h, z1_h, ones_h, degp, itemp, userp,
          acc_d, acc_i, acc_u, idx_v, ones_v, zbuf):
        c = lax.axis_index("c")
        s = lax.axis_index("s")
        pltpu.sync_copy(ones_h, ones_v)
        pltpu.sync_copy(z1_h, zbuf)
        _zero_1d(acc_d, zbuf, s)
        _zero_1d(acc_i, zbuf, s)
        _zero_1d(acc_u, zbuf, s)
        plsc.subcore_barrier()

        nblk = jnp.where(s < 5, 196, 195)
        sblk = s * 195 + jnp.minimum(s, 5)

        def dbody(kk, _):
            off = c * (_E // 2) + (sblk + kk) * 128
            pltpu.sync_copy(dst_h.at[pl.ds(off, 128)], idx_v.at[0])
            pltpu.sync_copy(ones_v, acc_d.at[idx_v.at[0]], add=True)
            return 0

        lax.fori_loop(0, nblk, dbody, 0)

        for t in range(2):
            off = (c * 32 + s * 2 + t) * 128
            pltpu.sync_copy(item_h.at[pl.ds(off, 128)], idx_v.at[0])
            pltpu.sync_copy(ones_v, acc_i.at[idx_v.at[0]], add=True)

        off = (c * 16 + s) * 128
        pltpu.sync_copy(user_h.at[pl.ds(off, 128)], idx_v.at[0])
        pltpu.sync_copy(ones_v, acc_u.at[idx_v.at[0]], add=True)

        plsc.subcore_barrier()
        _wb_1d(acc_d, degp, c, s, zbuf)
        _wb_1d(acc_i, itemp, c, s, zbuf)
        _wb_1d(acc_u, userp, c, s, zbuf)

    return k(dst, item_all, user_idx, zeros1, ones128)


def _sc_segsum(t0, t1, t2, t3, src, dst, zeros2):
    @functools.partial(
        pl.kernel,
        out_type=[jax.ShapeDtypeStruct((_N, 32), jnp.float32)] * 4,
        mesh=_sc_mesh(),
        compiler_params=pltpu.CompilerParams(use_tc_tiling_on_sc=False),
        scratch_types=[
            pltpu.VMEM_SHARED((_N, 32), jnp.float32),
            pltpu.VMEM((1, 128), jnp.int32),
            pltpu.VMEM((1, 128), jnp.int32),
            pltpu.VMEM((128, 32), jnp.float32),
            pltpu.VMEM((512, 32), jnp.float32),
            pltpu.SemaphoreType.DMA,
        ],
    )
    def k(t0_h, t1_h, t2_h, t3_h, src_h, dst_h, z2_h,
          s0_h, s1_h, s2_h, s3_h, acc, isrc, idst, rows_v, stg, sem):
        c = lax.axis_index("c")
        s = lax.axis_index("s")
        nblk = jnp.where(s < 10, 391, 390)
        sblk = s * 390 + jnp.minimum(s, 10)
        r0 = s * _NPT

        def process(t_h, out_h):
            pltpu.sync_copy(z2_h, stg)
            z0 = s * _WC

            def zbody(kk, _):
                pltpu.sync_copy(stg, acc.at[pl.ds(z0 + kk * 512, 512)])
                return 0

            lax.fori_loop(0, 6, zbody, 0)

            @pl.when(s < 15)
            def _():
                pltpu.sync_copy(stg.at[pl.ds(0, 56)], acc.at[pl.ds(z0 + 3072, 56)])

            @pl.when(s == 15)
            def _():
                pltpu.sync_copy(stg.at[pl.ds(0, 8)], acc.at[pl.ds(z0 + 3072, 8)])

            plsc.subcore_barrier()

            def body(kk, _):
                off = (sblk + kk) * 128
                pltpu.sync_copy(src_h.at[pl.ds(off, 128)], isrc.at[0])
                pltpu.async_copy(t_h.at[isrc.at[0]], rows_v, sem).wait()
                pltpu.sync_copy(dst_h.at[pl.ds(off, 128)], idst.at[0])
                pltpu.sync_copy(rows_v, acc.at[idst.at[0]], add=True)
                return 0

            lax.fori_loop(0, nblk, body, 0)
            plsc.subcore_barrier()

            w0 = s * _WC

            def wbody(kk, _):
                rr = w0 + kk * 512
                pltpu.sync_copy(acc.at[pl.ds(rr, 512)], stg)
                pltpu.sync_copy(stg, out_h.at[pl.ds(rr, 512)])
                return 0

            lax.fori_loop(0, 6, wbody, 0)
            rt = w0 + 3072

            @pl.when(s < 15)
            def _():
                pltpu.sync_copy(acc.at[pl.ds(rt, 56)], stg.at[pl.ds(0, 56)])
                pltpu.sync_copy(stg.at[pl.ds(0, 56)], out_h.at[pl.ds(rt, 56)])

            @pl.when(s == 15)
            def _():
                pltpu.sync_copy(acc.at[pl.ds(rt, 8)], stg.at[pl.ds(0, 8)])
                pltpu.sync_copy(stg.at[pl.ds(0, 8)], out_h.at[pl.ds(rt, 8)])

        @pl.when(c == 0)
        def _():
            process(t0_h, s0_h)
            process(t1_h, s1_h)

        @pl.when(c == 1)
        def _():
            process(t2_h, s2_h)
            process(t3_h, s3_h)

    return k(t0, t1, t2, t3, src, dst, zeros2)


def _sc_bgather(f, user_idx, itp, itn):
    @functools.partial(
        pl.kernel,
        out_type=[jax.ShapeDtypeStruct((_B, 128), jnp.float32)] * 3,
        mesh=_sc_mesh(),
        compiler_params=pltpu.CompilerParams(use_tc_tiling_on_sc=False),
        scratch_types=[
            pltpu.VMEM((1, 128), jnp.int32),
            pltpu.VMEM((128, 128), jnp.float32),
            pltpu.SemaphoreType.DMA,
        ],
    )
    def k(f_h, u_h, p_h, n_h, ur_h, pr_h, nr_h, idx_v, rows_v, sem):
        c = lax.axis_index("c")
        s = lax.axis_index("s")
        wid = s * 2 + c
        off = wid * 128
        for src_h, out_h in ((u_h, ur_h), (p_h, pr_h), (n_h, nr_h)):
            pltpu.sync_copy(src_h.at[pl.ds(off, 128)], idx_v.at[0])
            pltpu.async_copy(f_h.at[idx_v.at[0]], rows_v, sem).wait()
            pltpu.sync_copy(rows_v, out_h.at[pl.ds(off, 128)])

    return k(f, user_idx, itp, itn)


def _tc_prep(deg0, deg1, emb_int, emb_pop):
    def body(d0, d1, ei, ep, t0, t1, t2, t3, nrm):
        deg = jnp.maximum(d0[...] + d1[...], 1.0)
        r = lax.rsqrt(deg)
        nrm[...] = r
        t0[...] = ei[:, :32] * r
        t1[...] = ei[:, 32:] * r
        t2[...] = ep[:, :32] * r
        t3[...] = ep[:, 32:] * r

    return pl.pallas_call(
        body,
        grid=(_NB,),
        in_specs=[pl.BlockSpec((_BN, 1), lambda i: (i, 0))] * 2
        + [pl.BlockSpec((_BN, 64), lambda i: (i, 0))] * 2,
        out_specs=[pl.BlockSpec((_BN, 32), lambda i: (i, 0))] * 4
        + [pl.BlockSpec((_BN, 1), lambda i: (i, 0))],
        out_shape=[jax.ShapeDtypeStruct((_N, 32), jnp.float32)] * 4
        + [jax.ShapeDtypeStruct((_N, 1), jnp.float32)],
    )(deg0, deg1, emb_int, emb_pop)


def _tc_mid(s0, s1, s2, s3, nrm):
    def body(a0, a1, a2, a3, r, o0, o1, o2, o3):
        r2 = r[...] * r[...]
        o0[...] = a0[...] * r2
        o1[...] = a1[...] * r2
        o2[...] = a2[...] * r2
        o3[...] = a3[...] * r2

    return pl.pallas_call(
        body,
        grid=(_NB,),
        in_specs=[pl.BlockSpec((_BN, 32), lambda i: (i, 0))] * 4
        + [pl.BlockSpec((_BN, 1), lambda i: (i, 0))],
        out_specs=[pl.BlockSpec((_BN, 32), lambda i: (i, 0))] * 4,
        out_shape=[jax.ShapeDtypeStruct((_N, 32), jnp.float32)] * 4,
    )(s0, s1, s2, s3, nrm)


def _tc_feat(emb_int, emb_pop, s1, s2, nrm):
    def body(ei, ep, a0, a1, a2, a3, b0, b1, b2, b3, r, out):
        rr = r[...]
        third = jnp.float32(1.0 / 3.0)
        f0 = (ei[:, :32] + rr * (a0[...] + b0[...])) * third
        f1 = (ei[:, 32:] + rr * (a1[...] + b1[...])) * third
        f2 = (ep[:, :32] + rr * (a2[...] + b2[...])) * third
        f3 = (ep[:, 32:] + rr * (a3[...] + b3[...])) * third
        out[...] = jnp.concatenate([f0, f1, f2, f3], axis=1)

    return pl.pallas_call(
        body,
        grid=(_NB,),
        in_specs=[pl.BlockSpec((_BN, 64), lambda i: (i, 0))] * 2
        + [pl.BlockSpec((_BN, 32), lambda i: (i, 0))] * 8
        + [pl.BlockSpec((_BN, 1), lambda i: (i, 0))],
        out_specs=pl.BlockSpec((_BN, 128), lambda i: (i, 0)),
        out_shape=jax.ShapeDtypeStruct((_N, 128), jnp.float32),
    )(emb_int, emb_pop, *s1, *s2, nrm)


def _tc_loss(f, ci0, ci1, cu0, cu1, ur, pr, nr, maskf):
    def body(fb, i0, i1, u0, u1, u_r, p_r, n_r, mk, out, acc):
        step = pl.program_id(0)

        @pl.when(step == 0)
        def _():
            ui, up = u_r[:, :64], u_r[:, 64:]
            pi, pp = p_r[:, :64], p_r[:, 64:]
            ni_, np_ = n_r[:, :64], n_r[:, 64:]
            psi = jnp.sum(ui * pi, axis=1, keepdims=True)
            nsi = jnp.sum(ui * ni_, axis=1, keepdims=True)
            psp = jnp.sum(up * pp, axis=1, keepdims=True)
            nsp = jnp.sum(up * np_, axis=1, keepdims=True)
            m = mk[...]

            def lsig(x):
                return jnp.log(1.0 / (1.0 + jnp.exp(-x)))

            acc[0] = -jnp.mean(m * lsig(psi - nsi))
            acc[1] = (-jnp.mean(m * lsig(nsp - psp))
                      - jnp.mean((1.0 - m) * lsig(psp - nsp)))
            acc[2] = -jnp.mean(lsig(psi + psp - nsi - nsp))
            acc[3] = 0.0
            acc[4] = 0.0
            acc[5] = 0.0
            acc[6] = 0.0

        fi, fp = fb[:, :64], fb[:, 64:]
        rs = jnp.sum((fi - fp) ** 2, axis=1, keepdims=True)
        pres_i = (i0[...] + i1[...]) > 0.0
        pres_u = (u0[...] + u1[...]) > 0.0
        acc[3] = acc[3] + jnp.sum(jnp.where(pres_i, rs, 0.0))
        acc[4] = acc[4] + jnp.sum(pres_i.astype(jnp.float32))
        acc[5] = acc[5] + jnp.sum(jnp.where(pres_u, rs, 0.0))
        acc[6] = acc[6] + jnp.sum(pres_u.astype(jnp.float32))

        @pl.when(step == _NB - 1)
        def _():
            disc = acc[3] / (acc[4] * 64.0) + acc[5] / (acc[6] * 64.0)
            total = 0.1 * acc[0] + 0.1 * acc[1] + acc[2] - 0.01 * disc
            out[...] = jnp.broadcast_to(total, (1, 1))

    return pl.pallas_call(
        body,
        grid=(_NB,),
        in_specs=[pl.BlockSpec((_BN, 128), lambda i: (i, 0))]
        + [pl.BlockSpec((_BN, 1), lambda i: (i, 0))] * 4
        + [pl.BlockSpec((_B, 128), lambda i: (0, 0))] * 3
        + [pl.BlockSpec((_B, 1), lambda i: (0, 0))],
        out_specs=pl.BlockSpec((1, 1), lambda i: (0, 0)),
        out_shape=jax.ShapeDtypeStruct((1, 1), jnp.float32),
        scratch_shapes=[pltpu.SMEM((8,), jnp.float32)],
    )(f, ci0, ci1, cu0, cu1, ur, pr, nr, maskf)


def kernel(embeddings_int, embeddings_pop, user, item_p, item_n, mask, graph):
    src = graph[0]
    dst = graph[1]
    uidx = user.reshape(-1)
    itp = (item_p + _NU).reshape(-1)
    itn = (item_n + _NU).reshape(-1)
    item_all = jnp.concatenate([itp, itn])
    zeros1 = jnp.zeros((_ZC,), jnp.float32)
    zeros2 = jnp.zeros((512, 32), jnp.float32)
    ones128 = jnp.ones((128,), jnp.float32)

    degp, itemp, userp = _sc_stats(dst, item_all, uidx, zeros1, ones128)
    deg0 = degp[:_N].reshape(_N, 1)
    deg1 = degp[_NP:_NP + _N].reshape(_N, 1)
    *t, nrm = _tc_prep(deg0, deg1, embeddings_int, embeddings_pop)
    s1 = _sc_segsum(*t, src, dst, zeros2)
    t1 = _tc_mid(*s1, nrm)
    s2 = _sc_segsum(*t1, src, dst, zeros2)
    f = _tc_feat(embeddings_int, embeddings_pop, s1, s2, nrm)
    ur, pr, nr = _sc_bgather(f, uidx, itp, itn)

    ci0 = itemp[:_N].reshape(_N, 1)
    ci1 = itemp[_NP:_NP + _N].reshape(_N, 1)
    cu0 = userp[:_N].reshape(_N, 1)
    cu1 = userp[_NP:_NP + _N].reshape(_N, 1)
    maskf = mask.astype(jnp.float32)
    loss = _tc_loss(f, ci0, ci1, cu0, cu1, ur, pr, nr, maskf)
    return loss[0, 0]


# speedup vs baseline: 7.8315x; 2.5761x over previous
import functools

import jax
import jax.numpy as jnp
from jax import lax
from jax.experimental import pallas as pl
from jax.experimental.pallas import tpu as pltpu
from jax.experimental.pallas import tpu_sc as plsc

_NU = 25000
_N = 50000
_E = 800000
_B = 4096
_BN = 1000
_NB = _N // _BN
_NPT = _N // 16
_ZC = 3200
_NP = 16 * _ZC
_WC = 3128
_WL = _N - 15 * _WC

@functools.cache
def _sc_mesh():
    return plsc.VectorSubcoreMesh(core_axis_name="c", subcore_axis_name="s")


def _zero_1d(acc, zbuf, s):
    pltpu.sync_copy(zbuf, acc.at[pl.ds(s * _ZC, _ZC)])


def _wb_1d(acc, out, c, s, wbuf):
    off = s * _ZC
    pltpu.sync_copy(acc.at[pl.ds(off, _ZC)], wbuf)
    pltpu.sync_copy(wbuf, out.at[pl.ds(c * _NP + off, _ZC)])


def _sc_stats(dst, item_all, user_idx, zeros1, ones128):
    @functools.partial(
        pl.kernel,
        out_type=[jax.ShapeDtypeStruct((2 * _NP,), jnp.float32)] * 3,
        mesh=_sc_mesh(),
        compiler_params=pltpu.CompilerParams(use_tc_tiling_on_sc=False),
        scratch_types=[
            pltpu.VMEM_SHARED((_NP,), jnp.float32),
            pltpu.VMEM_SHARED((_NP,), jnp.float32),
            pltpu.VMEM_SHARED((_NP,), jnp.float32),
            pltpu.VMEM((1, 128), jnp.int32),
            pltpu.VMEM((128,), jnp.float32),
            pltpu.VMEM((_ZC,), jnp.float32),
        ],
    )
    def k(dst_h, item_h, user_h, z1_h, ones_h, degp, itemp, userp,
          acc_d, acc_i, acc_u, idx_v, ones_v, zbuf):
        c = lax.axis_index("c")
        s = lax.axis_index("s")
        pltpu.sync_copy(ones_h, ones_v)
        pltpu.sync_copy(z1_h, zbuf)
        _zero_1d(acc_d, zbuf, s)
        _zero_1d(acc_i, zbuf, s)
        _zero_1d(acc_u, zbuf, s)
        plsc.subcore_barrier()

        nblk = jnp.where(s < 5, 196, 195)
        sblk = s * 195 + jnp.minimum(s, 5)

        def dbody(kk, _):
            off = c * (_E // 2) + (sblk + kk) * 128
            pltpu.sync_copy(dst_h.at[pl.ds(off, 128)], idx_v.at[0])
            pltpu.sync_copy(ones_v, acc_d.at[idx_v.at[0]], add=True)
            return 0

        lax.fori_loop(0, nblk, dbody, 0)

        for t in range(2):
            off = (c * 32 + s * 2 + t) * 128
            pltpu.sync_copy(item_h.at[pl.ds(off, 128)], idx_v.at[0])
            pltpu.sync_copy(ones_v, acc_i.at[idx_v.at[0]], add=True)

        off = (c * 16 + s) * 128
        pltpu.sync_copy(user_h.at[pl.ds(off, 128)], idx_v.at[0])
        pltpu.sync_copy(ones_v, acc_u.at[idx_v.at[0]], add=True)

        plsc.subcore_barrier()
        _wb_1d(acc_d, degp, c, s, zbuf)
        _wb_1d(acc_i, itemp, c, s, zbuf)
        _wb_1d(acc_u, userp, c, s, zbuf)

    return k(dst, item_all, user_idx, zeros1, ones128)


def _sc_segsum(t0, t1, t2, t3, src2, dst2, zeros2):
    @functools.partial(
        pl.kernel,
        out_type=[jax.ShapeDtypeStruct((_N, 32), jnp.float32)] * 4,
        mesh=_sc_mesh(),
        compiler_params=pltpu.CompilerParams(use_tc_tiling_on_sc=False),
        scratch_types=[
            pltpu.VMEM_SHARED((_N, 32), jnp.float32),
            pltpu.VMEM((4, 128), jnp.int32),
            pltpu.VMEM((4, 128), jnp.int32),
            pltpu.VMEM((4, 128), jnp.int32),
            pltpu.VMEM((4, 128), jnp.int32),
            pltpu.VMEM((4, 128, 32), jnp.float32),
            pltpu.VMEM((256, 32), jnp.float32),
            pltpu.SemaphoreType.DMA,
            pltpu.SemaphoreType.DMA,
            pltpu.SemaphoreType.DMA,
            pltpu.SemaphoreType.DMA,
        ],
    )
    def k(t0_h, t1_h, t2_h, t3_h, src_h, dst_h, z2_h,
          s0_h, s1_h, s2_h, s3_h, acc, sa, da, sb, db, rows_v, stg,
          m0, m1, m2, m3):
        c = lax.axis_index("c")
        s = lax.axis_index("s")
        sems = (m0, m1, m2, m3)
        nblk = jnp.where(s < 10, 391, 390)
        sblk = s * 390 + jnp.minimum(s, 10)
        ng2 = 24
        nrem = nblk - 384

        def process(t_h, out_h):
            pltpu.sync_copy(z2_h, stg)
            z0 = s * _WC

            def zbody(kk, _):
                pltpu.sync_copy(stg, acc.at[pl.ds(z0 + kk * 256, 256)])
                return 0

            lax.fori_loop(0, 12, zbody, 0)

            @pl.when(s < 15)
            def _():
                pltpu.sync_copy(stg.at[pl.ds(0, 56)], acc.at[pl.ds(z0 + 3072, 56)])

            @pl.when(s == 15)
            def _():
                pltpu.sync_copy(stg.at[pl.ds(0, 8)], acc.at[pl.ds(z0 + 3072, 8)])

            plsc.subcore_barrier()

            def fire(sidx):
                return [
                    pltpu.async_copy(t_h.at[sidx.at[j]], rows_v.at[j], sems[j])
                    for j in range(4)
                ]

            def drain(descs, didx):
                for j in range(4):
                    descs[j].wait()
                    pltpu.sync_copy(rows_v.at[j], acc.at[didx.at[j]], add=True)

            pltpu.sync_copy(src_h.at[pl.ds(sblk, 4)], sa)

            def gbody(g, _):
                ra = sblk + g * 8
                descs = fire(sa)
                pltpu.sync_copy(dst_h.at[pl.ds(ra, 4)], da)
                pltpu.sync_copy(src_h.at[pl.ds(ra + 4, 4)], sb)
                drain(descs, da)
                descs = fire(sb)
                pltpu.sync_copy(dst_h.at[pl.ds(ra + 4, 4)], db)
                pltpu.sync_copy(src_h.at[pl.ds(ra + 8, 4)], sa)
                drain(descs, db)
                return 0

            lax.fori_loop(0, ng2, gbody, 0)

            def rbody(kk, _):
                bb = sblk + 384 + kk
                pltpu.sync_copy(src_h.at[pl.ds(bb, 1)], sa.at[pl.ds(0, 1)])
                pltpu.async_copy(t_h.at[sa.at[0]], rows_v.at[0], m0).wait()
                pltpu.sync_copy(dst_h.at[pl.ds(bb, 1)], da.at[pl.ds(0, 1)])
                pltpu.sync_copy(rows_v.at[0], acc.at[da.at[0]], add=True)
                return 0

            lax.fori_loop(0, nrem, rbody, 0)
            plsc.subcore_barrier()

            w0 = s * _WC

            def wbody(kk, _):
                rr = w0 + kk * 256
                pltpu.sync_copy(acc.at[pl.ds(rr, 256)], stg)
                pltpu.sync_copy(stg, out_h.at[pl.ds(rr, 256)])
                return 0

            lax.fori_loop(0, 12, wbody, 0)
            rt = w0 + 3072

            @pl.when(s < 15)
            def _():
                pltpu.sync_copy(acc.at[pl.ds(rt, 56)], stg.at[pl.ds(0, 56)])
                pltpu.sync_copy(stg.at[pl.ds(0, 56)], out_h.at[pl.ds(rt, 56)])

            @pl.when(s == 15)
            def _():
                pltpu.sync_copy(acc.at[pl.ds(rt, 8)], stg.at[pl.ds(0, 8)])
                pltpu.sync_copy(stg.at[pl.ds(0, 8)], out_h.at[pl.ds(rt, 8)])

        @pl.when(c == 0)
        def _():
            process(t0_h, s0_h)
            process(t1_h, s1_h)

        @pl.when(c == 1)
        def _():
            process(t2_h, s2_h)
            process(t3_h, s3_h)

    return k(t0, t1, t2, t3, src2, dst2, zeros2)


def _sc_bgather(f, user_idx, itp, itn):
    @functools.partial(
        pl.kernel,
        out_type=[jax.ShapeDtypeStruct((_B, 128), jnp.float32)] * 3,
        mesh=_sc_mesh(),
        compiler_params=pltpu.CompilerParams(use_tc_tiling_on_sc=False),
        scratch_types=[
            pltpu.VMEM((1, 128), jnp.int32),
            pltpu.VMEM((128, 128), jnp.float32),
            pltpu.SemaphoreType.DMA,
        ],
    )
    def k(f_h, u_h, p_h, n_h, ur_h, pr_h, nr_h, idx_v, rows_v, sem):
        c = lax.axis_index("c")
        s = lax.axis_index("s")
        wid = s * 2 + c
        off = wid * 128
        for src_h, out_h in ((u_h, ur_h), (p_h, pr_h), (n_h, nr_h)):
            pltpu.sync_copy(src_h.at[pl.ds(off, 128)], idx_v.at[0])
            pltpu.async_copy(f_h.at[idx_v.at[0]], rows_v, sem).wait()
            pltpu.sync_copy(rows_v, out_h.at[pl.ds(off, 128)])

    return k(f, user_idx, itp, itn)


def _tc_prep(deg0, deg1, emb_int, emb_pop):
    def body(d0, d1, ei, ep, t0, t1, t2, t3, nrm):
        deg = jnp.maximum(d0[...] + d1[...], 1.0)
        r = lax.rsqrt(deg)
        nrm[...] = r
        t0[...] = ei[:, :32] * r
        t1[...] = ei[:, 32:] * r
        t2[...] = ep[:, :32] * r
        t3[...] = ep[:, 32:] * r

    return pl.pallas_call(
        body,
        grid=(_NB,),
        in_specs=[pl.BlockSpec((_BN, 1), lambda i: (i, 0))] * 2
        + [pl.BlockSpec((_BN, 64), lambda i: (i, 0))] * 2,
        out_specs=[pl.BlockSpec((_BN, 32), lambda i: (i, 0))] * 4
        + [pl.BlockSpec((_BN, 1), lambda i: (i, 0))],
        out_shape=[jax.ShapeDtypeStruct((_N, 32), jnp.float32)] * 4
        + [jax.ShapeDtypeStruct((_N, 1), jnp.float32)],
    )(deg0, deg1, emb_int, emb_pop)


def _tc_mid(s0, s1, s2, s3, nrm):
    def body(a0, a1, a2, a3, r, o0, o1, o2, o3):
        r2 = r[...] * r[...]
        o0[...] = a0[...] * r2
        o1[...] = a1[...] * r2
        o2[...] = a2[...] * r2
        o3[...] = a3[...] * r2

    return pl.pallas_call(
        body,
        grid=(_NB,),
        in_specs=[pl.BlockSpec((_BN, 32), lambda i: (i, 0))] * 4
        + [pl.BlockSpec((_BN, 1), lambda i: (i, 0))],
        out_specs=[pl.BlockSpec((_BN, 32), lambda i: (i, 0))] * 4,
        out_shape=[jax.ShapeDtypeStruct((_N, 32), jnp.float32)] * 4,
    )(s0, s1, s2, s3, nrm)


def _tc_feat(emb_int, emb_pop, s1, s2, nrm):
    def body(ei, ep, a0, a1, a2, a3, b0, b1, b2, b3, r, out):
        rr = r[...]
        third = jnp.float32(1.0 / 3.0)
        f0 = (ei[:, :32] + rr * (a0[...] + b0[...])) * third
        f1 = (ei[:, 32:] + rr * (a1[...] + b1[...])) * third
        f2 = (ep[:, :32] + rr * (a2[...] + b2[...])) * third
        f3 = (ep[:, 32:] + rr * (a3[...] + b3[...])) * third
        out[...] = jnp.concatenate([f0, f1, f2, f3], axis=1)

    return pl.pallas_call(
        body,
        grid=(_NB,),
        in_specs=[pl.BlockSpec((_BN, 64), lambda i: (i, 0))] * 2
        + [pl.BlockSpec((_BN, 32), lambda i: (i, 0))] * 8
        + [pl.BlockSpec((_BN, 1), lambda i: (i, 0))],
        out_specs=pl.BlockSpec((_BN, 128), lambda i: (i, 0)),
        out_shape=jax.ShapeDtypeStruct((_N, 128), jnp.float32),
    )(emb_int, emb_pop, *s1, *s2, nrm)


def _tc_loss(f, ci0, ci1, cu0, cu1, ur, pr, nr, maskf):
    def body(fb, i0, i1, u0, u1, u_r, p_r, n_r, mk, out, acc):
        step = pl.program_id(0)

        @pl.when(step == 0)
        def _():
            ui, up = u_r[:, :64], u_r[:, 64:]
            pi, pp = p_r[:, :64], p_r[:, 64:]
            ni_, np_ = n_r[:, :64], n_r[:, 64:]
            psi = jnp.sum(ui * pi, axis=1, keepdims=True)
            nsi = jnp.sum(ui * ni_, axis=1, keepdims=True)
            psp = jnp.sum(up * pp, axis=1, keepdims=True)
            nsp = jnp.sum(up * np_, axis=1, keepdims=True)
            m = mk[...]

            def lsig(x):
                return jnp.log(1.0 / (1.0 + jnp.exp(-x)))

            acc[0] = -jnp.mean(m * lsig(psi - nsi))
            acc[1] = (-jnp.mean(m * lsig(nsp - psp))
                      - jnp.mean((1.0 - m) * lsig(psp - nsp)))
            acc[2] = -jnp.mean(lsig(psi + psp - nsi - nsp))
            acc[3] = 0.0
            acc[4] = 0.0
            acc[5] = 0.0
            acc[6] = 0.0

        fi, fp = fb[:, :64], fb[:, 64:]
        rs = jnp.sum((fi - fp) ** 2, axis=1, keepdims=True)
        pres_i = (i0[...] + i1[...]) > 0.0
        pres_u = (u0[...] + u1[...]) > 0.0
        acc[3] = acc[3] + jnp.sum(jnp.where(pres_i, rs, 0.0))
        acc[4] = acc[4] + jnp.sum(pres_i.astype(jnp.float32))
        acc[5] = acc[5] + jnp.sum(jnp.where(pres_u, rs, 0.0))
        acc[6] = acc[6] + jnp.sum(pres_u.astype(jnp.float32))

        @pl.when(step == _NB - 1)
        def _():
            disc = acc[3] / (acc[4] * 64.0) + acc[5] / (acc[6] * 64.0)
            total = 0.1 * acc[0] + 0.1 * acc[1] + acc[2] - 0.01 * disc
            out[...] = jnp.broadcast_to(total, (1, 1))

    return pl.pallas_call(
        body,
        grid=(_NB,),
        in_specs=[pl.BlockSpec((_BN, 128), lambda i: (i, 0))]
        + [pl.BlockSpec((_BN, 1), lambda i: (i, 0))] * 4
        + [pl.BlockSpec((_B, 128), lambda i: (0, 0))] * 3
        + [pl.BlockSpec((_B, 1), lambda i: (0, 0))],
        out_specs=pl.BlockSpec((1, 1), lambda i: (0, 0)),
        out_shape=jax.ShapeDtypeStruct((1, 1), jnp.float32),
        scratch_shapes=[pltpu.SMEM((8,), jnp.float32)],
    )(f, ci0, ci1, cu0, cu1, ur, pr, nr, maskf)


def kernel(embeddings_int, embeddings_pop, user, item_p, item_n, mask, graph):
    src = graph[0]
    dst = graph[1]
    uidx = user.reshape(-1)
    itp = (item_p + _NU).reshape(-1)
    itn = (item_n + _NU).reshape(-1)
    item_all = jnp.concatenate([itp, itn])
    zeros1 = jnp.zeros((_ZC,), jnp.float32)
    zeros2 = jnp.zeros((256, 32), jnp.float32)
    src2 = src.reshape(_E // 128, 128)
    dst2 = dst.reshape(_E // 128, 128)
    ones128 = jnp.ones((128,), jnp.float32)

    degp, itemp, userp = _sc_stats(dst, item_all, uidx, zeros1, ones128)
    deg0 = degp[:_N].reshape(_N, 1)
    deg1 = degp[_NP:_NP + _N].reshape(_N, 1)
    *t, nrm = _tc_prep(deg0, deg1, embeddings_int, embeddings_pop)
    s1 = _sc_segsum(*t, src2, dst2, zeros2)
    t1 = _tc_mid(*s1, nrm)
    s2 = _sc_segsum(*t1, src2, dst2, zeros2)
    f = _tc_feat(embeddings_int, embeddings_pop, s1, s2, nrm)
    ur, pr, nr = _sc_bgather(f, uidx, itp, itn)

    ci0 = itemp[:_N].reshape(_N, 1)
    ci1 = itemp[_NP:_NP + _N].reshape(_N, 1)
    cu0 = userp[:_N].reshape(_N, 1)
    cu1 = userp[_NP:_NP + _N].reshape(_N, 1)
    maskf = mask.astype(jnp.float32)
    loss = _tc_loss(f, ci0, ci1, cu0, cu1, ur, pr, nr, maskf)
    return loss[0, 0]


# speedup vs baseline: 8.0298x; 1.0253x over previous
import functools

import jax
import jax.numpy as jnp
from jax import lax
from jax.experimental import pallas as pl
from jax.experimental.pallas import tpu as pltpu
from jax.experimental.pallas import tpu_sc as plsc

_NU = 25000
_N = 50000
_E = 800000
_B = 4096
_BN = 1000
_NB = _N // _BN
_NPT = _N // 16
_ZC = 3200
_NP = 16 * _ZC
_WC = 3128
_WL = _N - 15 * _WC

@functools.cache
def _sc_mesh():
    return plsc.VectorSubcoreMesh(core_axis_name="c", subcore_axis_name="s")


def _zero_1d(acc, zbuf, s):
    pltpu.sync_copy(zbuf, acc.at[pl.ds(s * _ZC, _ZC)])


def _wb_1d(acc, out, c, s, wbuf):
    off = s * _ZC
    pltpu.sync_copy(acc.at[pl.ds(off, _ZC)], wbuf)
    pltpu.sync_copy(wbuf, out.at[pl.ds(c * _NP + off, _ZC)])


def _sc_stats(dst, item_all, user_idx, zeros1, ones128):
    @functools.partial(
        pl.kernel,
        out_type=[jax.ShapeDtypeStruct((2 * _NP,), jnp.float32)] * 3,
        mesh=_sc_mesh(),
        compiler_params=pltpu.CompilerParams(use_tc_tiling_on_sc=False),
        scratch_types=[
            pltpu.VMEM_SHARED((_NP,), jnp.float32),
            pltpu.VMEM_SHARED((_NP,), jnp.float32),
            pltpu.VMEM_SHARED((_NP,), jnp.float32),
            pltpu.VMEM((1, 128), jnp.int32),
            pltpu.VMEM((128,), jnp.float32),
            pltpu.VMEM((_ZC,), jnp.float32),
        ],
    )
    def k(dst_h, item_h, user_h, z1_h, ones_h, degp, itemp, userp,
          acc_d, acc_i, acc_u, idx_v, ones_v, zbuf):
        c = lax.axis_index("c")
        s = lax.axis_index("s")
        pltpu.sync_copy(ones_h, ones_v)
        pltpu.sync_copy(z1_h, zbuf)
        _zero_1d(acc_d, zbuf, s)
        _zero_1d(acc_i, zbuf, s)
        _zero_1d(acc_u, zbuf, s)
        plsc.subcore_barrier()

        nblk = jnp.where(s < 5, 196, 195)
        sblk = s * 195 + jnp.minimum(s, 5)

        def dbody(kk, _):
            off = c * (_E // 2) + (sblk + kk) * 128
            pltpu.sync_copy(dst_h.at[pl.ds(off, 128)], idx_v.at[0])
            pltpu.sync_copy(ones_v, acc_d.at[idx_v.at[0]], add=True)
            return 0

        lax.fori_loop(0, nblk, dbody, 0)

        for t in range(2):
            off = (c * 32 + s * 2 + t) * 128
            pltpu.sync_copy(item_h.at[pl.ds(off, 128)], idx_v.at[0])
            pltpu.sync_copy(ones_v, acc_i.at[idx_v.at[0]], add=True)

        off = (c * 16 + s) * 128
        pltpu.sync_copy(user_h.at[pl.ds(off, 128)], idx_v.at[0])
        pltpu.sync_copy(ones_v, acc_u.at[idx_v.at[0]], add=True)

        plsc.subcore_barrier()
        _wb_1d(acc_d, degp, c, s, zbuf)
        _wb_1d(acc_i, itemp, c, s, zbuf)
        _wb_1d(acc_u, userp, c, s, zbuf)

    return k(dst, item_all, user_idx, zeros1, ones128)


def _sc_segsum(t0, t1, t2, t3, src2, dst2, zeros2):
    @functools.partial(
        pl.kernel,
        out_type=[jax.ShapeDtypeStruct((_N, 32), jnp.float32)] * 4,
        mesh=_sc_mesh(),
        compiler_params=pltpu.CompilerParams(use_tc_tiling_on_sc=False),
        scratch_types=[
            pltpu.VMEM_SHARED((_N, 32), jnp.float32),
            pltpu.VMEM((4, 128), jnp.int32),
            pltpu.VMEM((4, 128), jnp.int32),
            pltpu.VMEM((4, 128), jnp.int32),
            pltpu.VMEM((4, 128), jnp.int32),
            pltpu.VMEM((4, 128, 32), jnp.float32),
            pltpu.VMEM((256, 32), jnp.float32),
            pltpu.SemaphoreType.DMA,
            pltpu.SemaphoreType.DMA,
            pltpu.SemaphoreType.DMA,
            pltpu.SemaphoreType.DMA,
        ],
    )
    def k(t0_h, t1_h, t2_h, t3_h, src_h, dst_h, z2_h,
          s0_h, s1_h, s2_h, s3_h, acc, sa, da, sb, db, rows_v, stg,
          m0, m1, m2, m3):
        c = lax.axis_index("c")
        s = lax.axis_index("s")
        sems = (m0, m1, m2, m3)
        nblk = jnp.where(s < 10, 391, 390)
        sblk = s * 390 + jnp.minimum(s, 10)
        ng2 = 24
        nrem = nblk - 384

        def process(t_h, out_h):
            pltpu.sync_copy(z2_h, stg)
            z0 = s * _WC

            def zbody(kk, _):
                pltpu.sync_copy(stg, acc.at[pl.ds(z0 + kk * 256, 256)])
                return 0

            lax.fori_loop(0, 12, zbody, 0)

            @pl.when(s < 15)
            def _():
                pltpu.sync_copy(stg.at[pl.ds(0, 56)], acc.at[pl.ds(z0 + 3072, 56)])

            @pl.when(s == 15)
            def _():
                pltpu.sync_copy(stg.at[pl.ds(0, 8)], acc.at[pl.ds(z0 + 3072, 8)])

            plsc.subcore_barrier()

            def fire(sidx):
                return [
                    pltpu.async_copy(t_h.at[sidx.at[j]], rows_v.at[j], sems[j])
                    for j in range(4)
                ]

            def drain(descs, didx):
                scat = []
                for j in range(4):
                    descs[j].wait()
                    scat.append(pltpu.async_copy(
                        rows_v.at[j], acc.at[didx.at[j]], sems[j], add=True))
                for d in scat:
                    d.wait()

            pltpu.sync_copy(src_h.at[pl.ds(sblk, 4)], sa)

            def gbody(g, _):
                ra = sblk + g * 8
                descs = fire(sa)
                pltpu.sync_copy(dst_h.at[pl.ds(ra, 4)], da)
                pltpu.sync_copy(src_h.at[pl.ds(ra + 4, 4)], sb)
                drain(descs, da)
                descs = fire(sb)
                pltpu.sync_copy(dst_h.at[pl.ds(ra + 4, 4)], db)
                pltpu.sync_copy(src_h.at[pl.ds(ra + 8, 4)], sa)
                drain(descs, db)
                return 0

            lax.fori_loop(0, ng2, gbody, 0)

            def rbody(kk, _):
                bb = sblk + 384 + kk
                pltpu.sync_copy(src_h.at[pl.ds(bb, 1)], sa.at[pl.ds(0, 1)])
                pltpu.async_copy(t_h.at[sa.at[0]], rows_v.at[0], m0).wait()
                pltpu.sync_copy(dst_h.at[pl.ds(bb, 1)], da.at[pl.ds(0, 1)])
                pltpu.sync_copy(rows_v.at[0], acc.at[da.at[0]], add=True)
                return 0

            lax.fori_loop(0, nrem, rbody, 0)
            plsc.subcore_barrier()

            w0 = s * _WC

            def wbody(kk, _):
                rr = w0 + kk * 256
                pltpu.sync_copy(acc.at[pl.ds(rr, 256)], stg)
                pltpu.sync_copy(stg, out_h.at[pl.ds(rr, 256)])
                return 0

            lax.fori_loop(0, 12, wbody, 0)
            rt = w0 + 3072

            @pl.when(s < 15)
            def _():
                pltpu.sync_copy(acc.at[pl.ds(rt, 56)], stg.at[pl.ds(0, 56)])
                pltpu.sync_copy(stg.at[pl.ds(0, 56)], out_h.at[pl.ds(rt, 56)])

            @pl.when(s == 15)
            def _():
                pltpu.sync_copy(acc.at[pl.ds(rt, 8)], stg.at[pl.ds(0, 8)])
                pltpu.sync_copy(stg.at[pl.ds(0, 8)], out_h.at[pl.ds(rt, 8)])

        @pl.when(c == 0)
        def _():
            process(t0_h, s0_h)
            process(t1_h, s1_h)

        @pl.when(c == 1)
        def _():
            process(t2_h, s2_h)
            process(t3_h, s3_h)

    return k(t0, t1, t2, t3, src2, dst2, zeros2)


def _sc_bgather(f, user_idx, itp, itn):
    @functools.partial(
        pl.kernel,
        out_type=[jax.ShapeDtypeStruct((_B, 128), jnp.float32)] * 3,
        mesh=_sc_mesh(),
        compiler_params=pltpu.CompilerParams(use_tc_tiling_on_sc=False),
        scratch_types=[
            pltpu.VMEM((1, 128), jnp.int32),
            pltpu.VMEM((128, 128), jnp.float32),
            pltpu.SemaphoreType.DMA,
        ],
    )
    def k(f_h, u_h, p_h, n_h, ur_h, pr_h, nr_h, idx_v, rows_v, sem):
        c = lax.axis_index("c")
        s = lax.axis_index("s")
        wid = s * 2 + c
        off = wid * 128
        for src_h, out_h in ((u_h, ur_h), (p_h, pr_h), (n_h, nr_h)):
            pltpu.sync_copy(src_h.at[pl.ds(off, 128)], idx_v.at[0])
            pltpu.async_copy(f_h.at[idx_v.at[0]], rows_v, sem).wait()
            pltpu.sync_copy(rows_v, out_h.at[pl.ds(off, 128)])

    return k(f, user_idx, itp, itn)


def _tc_prep(deg0, deg1, emb_int, emb_pop):
    def body(d0, d1, ei, ep, t0, t1, t2, t3, nrm):
        deg = jnp.maximum(d0[...] + d1[...], 1.0)
        r = lax.rsqrt(deg)
        nrm[...] = r
        t0[...] = ei[:, :32] * r
        t1[...] = ei[:, 32:] * r
        t2[...] = ep[:, :32] * r
        t3[...] = ep[:, 32:] * r

    return pl.pallas_call(
        body,
        grid=(_NB,),
        in_specs=[pl.BlockSpec((_BN, 1), lambda i: (i, 0))] * 2
        + [pl.BlockSpec((_BN, 64), lambda i: (i, 0))] * 2,
        out_specs=[pl.BlockSpec((_BN, 32), lambda i: (i, 0))] * 4
        + [pl.BlockSpec((_BN, 1), lambda i: (i, 0))],
        out_shape=[jax.ShapeDtypeStruct((_N, 32), jnp.float32)] * 4
        + [jax.ShapeDtypeStruct((_N, 1), jnp.float32)],
    )(deg0, deg1, emb_int, emb_pop)


def _tc_mid(s0, s1, s2, s3, nrm):
    def body(a0, a1, a2, a3, r, o0, o1, o2, o3):
        r2 = r[...] * r[...]
        o0[...] = a0[...] * r2
        o1[...] = a1[...] * r2
        o2[...] = a2[...] * r2
        o3[...] = a3[...] * r2

    return pl.pallas_call(
        body,
        grid=(_NB,),
        in_specs=[pl.BlockSpec((_BN, 32), lambda i: (i, 0))] * 4
        + [pl.BlockSpec((_BN, 1), lambda i: (i, 0))],
        out_specs=[pl.BlockSpec((_BN, 32), lambda i: (i, 0))] * 4,
        out_shape=[jax.ShapeDtypeStruct((_N, 32), jnp.float32)] * 4,
    )(s0, s1, s2, s3, nrm)


def _tc_feat(emb_int, emb_pop, s1, s2, nrm):
    def body(ei, ep, a0, a1, a2, a3, b0, b1, b2, b3, r, out):
        rr = r[...]
        third = jnp.float32(1.0 / 3.0)
        f0 = (ei[:, :32] + rr * (a0[...] + b0[...])) * third
        f1 = (ei[:, 32:] + rr * (a1[...] + b1[...])) * third
        f2 = (ep[:, :32] + rr * (a2[...] + b2[...])) * third
        f3 = (ep[:, 32:] + rr * (a3[...] + b3[...])) * third
        out[...] = jnp.concatenate([f0, f1, f2, f3], axis=1)

    return pl.pallas_call(
        body,
        grid=(_NB,),
        in_specs=[pl.BlockSpec((_BN, 64), lambda i: (i, 0))] * 2
        + [pl.BlockSpec((_BN, 32), lambda i: (i, 0))] * 8
        + [pl.BlockSpec((_BN, 1), lambda i: (i, 0))],
        out_specs=pl.BlockSpec((_BN, 128), lambda i: (i, 0)),
        out_shape=jax.ShapeDtypeStruct((_N, 128), jnp.float32),
    )(emb_int, emb_pop, *s1, *s2, nrm)


def _tc_loss(f, ci0, ci1, cu0, cu1, ur, pr, nr, maskf):
    def body(fb, i0, i1, u0, u1, u_r, p_r, n_r, mk, out, acc):
        step = pl.program_id(0)

        @pl.when(step == 0)
        def _():
            ui, up = u_r[:, :64], u_r[:, 64:]
            pi, pp = p_r[:, :64], p_r[:, 64:]
            ni_, np_ = n_r[:, :64], n_r[:, 64:]
            psi = jnp.sum(ui * pi, axis=1, keepdims=True)
            nsi = jnp.sum(ui * ni_, axis=1, keepdims=True)
            psp = jnp.sum(up * pp, axis=1, keepdims=True)
            nsp = jnp.sum(up * np_, axis=1, keepdims=True)
            m = mk[...]

            def lsig(x):
                return jnp.log(1.0 / (1.0 + jnp.exp(-x)))

            acc[0] = -jnp.mean(m * lsig(psi - nsi))
            acc[1] = (-jnp.mean(m * lsig(nsp - psp))
                      - jnp.mean((1.0 - m) * lsig(psp - nsp)))
            acc[2] = -jnp.mean(lsig(psi + psp - nsi - nsp))
            acc[3] = 0.0
            acc[4] = 0.0
            acc[5] = 0.0
            acc[6] = 0.0

        fi, fp = fb[:, :64], fb[:, 64:]
        rs = jnp.sum((fi - fp) ** 2, axis=1, keepdims=True)
        pres_i = (i0[...] + i1[...]) > 0.0
        pres_u = (u0[...] + u1[...]) > 0.0
        acc[3] = acc[3] + jnp.sum(jnp.where(pres_i, rs, 0.0))
        acc[4] = acc[4] + jnp.sum(pres_i.astype(jnp.float32))
        acc[5] = acc[5] + jnp.sum(jnp.where(pres_u, rs, 0.0))
        acc[6] = acc[6] + jnp.sum(pres_u.astype(jnp.float32))

        @pl.when(step == _NB - 1)
        def _():
            disc = acc[3] / (acc[4] * 64.0) + acc[5] / (acc[6] * 64.0)
            total = 0.1 * acc[0] + 0.1 * acc[1] + acc[2] - 0.01 * disc
            out[...] = jnp.broadcast_to(total, (1, 1))

    return pl.pallas_call(
        body,
        grid=(_NB,),
        in_specs=[pl.BlockSpec((_BN, 128), lambda i: (i, 0))]
        + [pl.BlockSpec((_BN, 1), lambda i: (i, 0))] * 4
        + [pl.BlockSpec((_B, 128), lambda i: (0, 0))] * 3
        + [pl.BlockSpec((_B, 1), lambda i: (0, 0))],
        out_specs=pl.BlockSpec((1, 1), lambda i: (0, 0)),
        out_shape=jax.ShapeDtypeStruct((1, 1), jnp.float32),
        scratch_shapes=[pltpu.SMEM((8,), jnp.float32)],
    )(f, ci0, ci1, cu0, cu1, ur, pr, nr, maskf)


def kernel(embeddings_int, embeddings_pop, user, item_p, item_n, mask, graph):
    src = graph[0]
    dst = graph[1]
    uidx = user.reshape(-1)
    itp = (item_p + _NU).reshape(-1)
    itn = (item_n + _NU).reshape(-1)
    item_all = jnp.concatenate([itp, itn])
    zeros1 = jnp.zeros((_ZC,), jnp.float32)
    zeros2 = jnp.zeros((256, 32), jnp.float32)
    src2 = src.reshape(_E // 128, 128)
    dst2 = dst.reshape(_E // 128, 128)
    ones128 = jnp.ones((128,), jnp.float32)

    degp, itemp, userp = _sc_stats(dst, item_all, uidx, zeros1, ones128)
    deg0 = degp[:_N].reshape(_N, 1)
    deg1 = degp[_NP:_NP + _N].reshape(_N, 1)
    *t, nrm = _tc_prep(deg0, deg1, embeddings_int, embeddings_pop)
    s1 = _sc_segsum(*t, src2, dst2, zeros2)
    t1 = _tc_mid(*s1, nrm)
    s2 = _sc_segsum(*t1, src2, dst2, zeros2)
    f = _tc_feat(embeddings_int, embeddings_pop, s1, s2, nrm)
    ur, pr, nr = _sc_bgather(f, uidx, itp, itn)

    ci0 = itemp[:_N].reshape(_N, 1)
    ci1 = itemp[_NP:_NP + _N].reshape(_N, 1)
    cu0 = userp[:_N].reshape(_N, 1)
    cu1 = userp[_NP:_NP + _N].reshape(_N, 1)
    maskf = mask.astype(jnp.float32)
    loss = _tc_loss(f, ci0, ci1, cu0, cu1, ur, pr, nr, maskf)
    return loss[0, 0]
